# baseline (speedup 1.0000x reference)
import functools

import jax
import jax.numpy as jnp
from jax import lax
from jax.experimental import pallas as pl
from jax.experimental.pallas import tpu as pltpu
from jax.experimental.pallas import tpu_sc as plsc

B0 = 4096
B1 = 200
EMB = 64
VOCAB = 1000000
SCALE = 8.0

TR = B1 // 8
TC = B0 // 128
NTILES = TR * TC

NC = 2
NS = 16
NW = NC * NS
TPW = NTILES // NW
NSLOT = 8
GDEPTH = 4

_mesh = plsc.VectorSubcoreMesh(
    core_axis_name="c", subcore_axis_name="s", num_cores=NC, num_subcores=NS
)


_WBLK = 512


def _w_relayout_body(wt_ref, out_ref):
    blk = wt_ref[...]
    xp = blk.T
    pairs = xp.reshape(_WBLK // 2, 2, EMB)
    out_ref[...] = jnp.concatenate([pairs[:, 0, :], pairs[:, 1, :]], axis=1)


_w_relayout = pl.pallas_call(
    _w_relayout_body,
    grid=((VOCAB + _WBLK - 1) // _WBLK,),
    in_specs=[pl.BlockSpec((EMB, _WBLK), lambda i: (0, i))],
    out_specs=pl.BlockSpec((_WBLK // 2, 2 * EMB), lambda i: (i, 0)),
    out_shape=jax.ShapeDtypeStruct((VOCAB // 2, 2 * EMB), jnp.float32),
)


@functools.partial(
    pl.kernel,
    out_type=jax.ShapeDtypeStruct((B0, B1 * EMB), jnp.float32),
    mesh=_mesh,
    scratch_types=(
        [pltpu.VMEM((TPW, 8, 128), jnp.int32)]
        + [pltpu.VMEM((NSLOT, 128, EMB), jnp.float32)]
        + [pltpu.SemaphoreType.DMA] * (2 * NSLOT)
    ),
    compiler_params=pltpu.CompilerParams(use_tc_tiling_on_sc=False),
)
def _gather(idx_hbm, table_hbm, out_hbm, idx_all, rows_v, *sems):
    gsems = sems[:NSLOT]
    osems = sems[NSLOT:]
    wid = lax.axis_index("s") * NC + lax.axis_index("c")
    t0 = wid * TPW

    pltpu.sync_copy(idx_hbm.at[pl.ds(t0, TPW)], idx_all)

    def gather_copy(g, r):
        return pltpu.make_async_copy(
            table_hbm.at[idx_all.at[g, r]], rows_v.at[r], gsems[r]
        )

    def out_copy(g, r):
        t_id = t0 + g
        tc = lax.rem(t_id, TC)
        tr = lax.div(t_id, TC)
        return pltpu.make_async_copy(
            rows_v.at[r],
            out_hbm.at[pl.ds(tc * 128, 128), pl.ds((tr * 8 + r) * EMB, EMB)],
            osems[r],
        )

    def scale_slot(r):
        rv = rows_v.at[r]

        def sbody(row2, _):
            for u in range(2):
                row = row2 * 2 + u
                for k in range(EMB // 16):
                    sl = pl.ds(k * 16, 16)
                    rv[row, sl] = rv[row, sl] * SCALE
            return 0

        lax.fori_loop(0, 128 // 2, sbody, 0, unroll=2)

    for r in range(GDEPTH):
        gather_copy(0, r).start()

    def outer(g, _):
        for r in range(NSLOT):
            gather_copy(g, r).wait()
            scale_slot(r)
            out_copy(g, r).start()
            if r < GDEPTH:
                @pl.when(g >= 1)
                def _():
                    out_copy(g - 1, r + GDEPTH).wait()

                gather_copy(g, r + GDEPTH).start()
            else:
                @pl.when(g + 1 < TPW)
                def _():
                    out_copy(g, r - GDEPTH).wait()
                    gather_copy(g + 1, r - GDEPTH).start()

        return 0

    lax.fori_loop(0, TPW, outer, 0)

    for r in range(NSLOT):
        out_copy(TPW - 1, r).wait()


def _out_relayout_body(in_ref, out_ref):
    blk = in_ref[...]
    xp = blk.T
    out_ref[...] = xp.reshape(2, EMB, 512)


_out_relayout = pl.pallas_call(
    _out_relayout_body,
    grid=(B1 // 2, B0 // 512),
    in_specs=[pl.BlockSpec((512, 128), lambda i, j: (j, i))],
    out_specs=pl.BlockSpec((2, EMB, 512), lambda i, j: (i, 0, j)),
    out_shape=jax.ShapeDtypeStruct((B1, EMB, B0), jnp.float32),
)


def kernel(x, W):
    xv = (
        x.T.reshape(TR, 8, TC, 128)
        .transpose(0, 2, 1, 3)
        .reshape(NTILES, 8, 128)
        .astype(jnp.int32)
    )
    table = _w_relayout(W.T).reshape(VOCAB, EMB)
    flat = _gather(xv, table)
    out_t = _out_relayout(flat)
    return out_t.transpose(2, 0, 1)


# speedup vs baseline: 2.8764x; 2.8764x over previous
import functools

import jax
import jax.numpy as jnp
from jax import lax
from jax.experimental import pallas as pl
from jax.experimental.pallas import tpu as pltpu
from jax.experimental.pallas import tpu_sc as plsc

B0 = 4096
B1 = 200
EMB = 64
VOCAB = 1000000
SCALE = 8.0

TR = B1 // 8
TC = B0 // 128
NTILES = TR * TC

NC = 2
NS = 16
NW = NC * NS
TPW = NTILES // NW
NSLOT = 8
GDEPTH = 4

_mesh = plsc.VectorSubcoreMesh(
    core_axis_name="c", subcore_axis_name="s", num_cores=NC, num_subcores=NS
)


_WBLK = 4096


def _w_relayout_body(wt_ref, out_ref):
    blk = wt_ref[...]
    xp = blk.T
    pairs = xp.reshape(_WBLK // 2, 2, EMB)
    out_ref[...] = jnp.concatenate([pairs[:, 0, :], pairs[:, 1, :]], axis=1)


_w_relayout = pl.pallas_call(
    _w_relayout_body,
    grid=((VOCAB + _WBLK - 1) // _WBLK,),
    in_specs=[pl.BlockSpec((EMB, _WBLK), lambda i: (0, i))],
    out_specs=pl.BlockSpec((_WBLK // 2, 2 * EMB), lambda i: (i, 0)),
    out_shape=jax.ShapeDtypeStruct((VOCAB // 2, 2 * EMB), jnp.float32),
)


@functools.partial(
    pl.kernel,
    out_type=jax.ShapeDtypeStruct((B1 // 2, B0, 2 * EMB), jnp.float32),
    mesh=_mesh,
    scratch_types=(
        [pltpu.VMEM((TPW, 8, 128), jnp.int32)]
        + [pltpu.VMEM((NSLOT, 128, EMB), jnp.float32)]
        + [pltpu.SemaphoreType.DMA] * (2 * NSLOT)
    ),
    compiler_params=pltpu.CompilerParams(use_tc_tiling_on_sc=False),
)
def _gather(idx_hbm, table_hbm, out_hbm, idx_all, rows_v, *sems):
    gsems = sems[:NSLOT]
    osems = sems[NSLOT:]
    wid = lax.axis_index("s") * NC + lax.axis_index("c")
    t0 = wid * TPW

    pltpu.sync_copy(idx_hbm.at[pl.ds(t0, TPW)], idx_all)

    def gather_copy(g, r):
        return pltpu.make_async_copy(
            table_hbm.at[idx_all.at[g, r]], rows_v.at[r], gsems[r]
        )

    def out_copy(g, r):
        t_id = t0 + g
        tc = lax.rem(t_id, TC)
        tr = lax.div(t_id, TC)
        return pltpu.make_async_copy(
            rows_v.at[r],
            out_hbm.at[
                tr * 4 + (r // 2), pl.ds(tc * 128, 128), pl.ds((r % 2) * EMB, EMB)
            ],
            osems[r],
        )

    def scale_slot(r):
        rv = rows_v.at[r]

        def sbody(row2, _):
            for u in range(2):
                row = row2 * 2 + u
                for k in range(EMB // 16):
                    sl = pl.ds(k * 16, 16)
                    rv[row, sl] = rv[row, sl] * SCALE
            return 0

        lax.fori_loop(0, 128 // 2, sbody, 0, unroll=2)

    for r in range(GDEPTH):
        gather_copy(0, r).start()

    def outer(g, _):
        for r in range(NSLOT):
            gather_copy(g, r).wait()
            scale_slot(r)
            out_copy(g, r).start()
            if r < GDEPTH:
                @pl.when(g >= 1)
                def _():
                    out_copy(g - 1, r + GDEPTH).wait()

                gather_copy(g, r + GDEPTH).start()
            else:
                @pl.when(g + 1 < TPW)
                def _():
                    out_copy(g, r - GDEPTH).wait()
                    gather_copy(g + 1, r - GDEPTH).start()

        return 0

    lax.fori_loop(0, TPW, outer, 0)

    for r in range(NSLOT):
        out_copy(TPW - 1, r).wait()


def _out_relayout_body(in_ref, out_ref):
    blk = in_ref[...]
    out_ref[...] = blk.T.reshape(2, EMB, B0)


_out_relayout = pl.pallas_call(
    _out_relayout_body,
    grid=(B1 // 2,),
    in_specs=[pl.BlockSpec((B0, 2 * EMB), lambda j: (j, 0))],
    out_specs=pl.BlockSpec((2, EMB, B0), lambda j: (j, 0, 0)),
    out_shape=jax.ShapeDtypeStruct((B1, EMB, B0), jnp.float32),
)


def kernel(x, W):
    xv = (
        x.T.reshape(TR, 8, TC, 128)
        .transpose(0, 2, 1, 3)
        .reshape(NTILES, 8, 128)
        .astype(jnp.int32)
    )
    table = _w_relayout(W.T).reshape(VOCAB, EMB)
    flat = _gather(xv, table).reshape((B1 // 2) * B0, 2 * EMB)
    out_t = _out_relayout(flat)
    return out_t.transpose(2, 0, 1)
